# Initial kernel scaffold; baseline (speedup 1.0000x reference)
import functools

import jax
import jax.numpy as jnp
from jax import lax
from jax.experimental import pallas as pl
from jax.experimental.pallas import tpu as pltpu
from jax.experimental.pallas import tpu_sc as plsc

N = 10000
E = 160000
R = 16
NBASE = 4
D = 256
BATCH = 2
NP = 10240
BLK = 512

NC = 2
NS = 16
LANES = 16
EPS = E // NS
CH = 80
NCH = EPS // CH
STRIPE = N // NS


def _wbuild_body(w_comp_ref, basis_ref, loop_ref, out_ref):
    w = jnp.dot(w_comp_ref[...], basis_ref[...],
                preferred_element_type=jnp.float32)
    out_ref[0:R, :] = w
    out_ref[R:R + 1, :] = loop_ref[...].reshape(1, D * D)


def _build_wall(w_comp, basis, loop_w):
    out = pl.pallas_call(
        _wbuild_body,
        out_shape=jax.ShapeDtypeStruct((R + 1, D * D), jnp.float32),
    )(w_comp, basis.reshape(NBASE, D * D), loop_w.reshape(D, D))
    return out.reshape(R + 1, D, D)


def _proj_body(h_ref, w_ref, out_ref):
    out_ref[0, 0] = jnp.dot(h_ref[0], w_ref[0],
                            preferred_element_type=jnp.float32)


def _project(h_pad, wall):
    grid = (BATCH, NP // BLK, R + 1)
    return pl.pallas_call(
        _proj_body,
        grid=grid,
        in_specs=[
            pl.BlockSpec((1, BLK, D), lambda b, n, r: (b, n, 0)),
            pl.BlockSpec((1, D, D), lambda b, n, r: (r, 0, 0)),
        ],
        out_specs=pl.BlockSpec((1, 1, BLK, D), lambda b, n, r: (b, r, n, 0)),
        out_shape=jax.ShapeDtypeStruct((BATCH, R + 1, NP, D), jnp.float32),
    )(h_pad, wall)


def _final_body(agg_ref, hall_ref, out_ref, *, relu):
    o = agg_ref[0] + hall_ref[0, 0]
    if relu:
        o = jnp.maximum(o, 0.0)
    out_ref[0] = o


def _finalize(agg, hall, relu):
    grid = (BATCH, NP // BLK)
    return pl.pallas_call(
        functools.partial(_final_body, relu=relu),
        grid=grid,
        in_specs=[
            pl.BlockSpec((1, BLK, D), lambda b, n: (b, n, 0)),
            pl.BlockSpec((1, 1, BLK, D), lambda b, n: (b, R, n, 0)),
        ],
        out_specs=pl.BlockSpec((1, BLK, D), lambda b, n: (b, n, 0)),
        out_shape=jax.ShapeDtypeStruct((BATCH, NP, D), jnp.float32),
    )(agg, hall)


def _sc_body(hall2, srcg, etg, normg, dstg, zrows, out,
             src_v, et_v, norm_v, dst_v, gidx_v, rows_v, agg_sh, sem):
    c = lax.axis_index("c")
    s = lax.axis_index("s")

    pltpu.sync_copy(srcg.at[s], src_v)
    pltpu.sync_copy(etg.at[s], et_v)
    pltpu.sync_copy(normg.at[s], norm_v)
    pltpu.sync_copy(dstg.at[s], dst_v)

    for bt in range(BATCH):
        pltpu.sync_copy(zrows.at[pl.ds(s * STRIPE, STRIPE)],
                        agg_sh.at[pl.ds(s * STRIPE, STRIPE)])
        plsc.subcore_barrier()

        def chunk(k, carry):
            base = k * CH
            for j in range(CH // LANES):
                et = et_v[pl.ds(base + j * LANES, LANES)]
                sr = src_v[pl.ds(base + j * LANES, LANES)]
                gidx_v[pl.ds(j * LANES, LANES)] = (
                    ((bt * (R + 1) + et) * NP + sr) * 2 + c)
            pltpu.async_copy(hall2.at[gidx_v], rows_v, sem).wait()
            for e in range(CH):
                nv = plsc.load_gather(
                    norm_v, [jnp.full((LANES,), base + e, jnp.int32)])
                for q in range(D // 2 // LANES):
                    rows_v[e, pl.ds(q * LANES, LANES)] = (
                        rows_v[e, pl.ds(q * LANES, LANES)] * nv)
            pltpu.sync_copy(rows_v, agg_sh.at[dst_v.at[k]], add=True)
            return carry

        lax.fori_loop(0, NCH, chunk, 0)
        plsc.subcore_barrier()
        pltpu.sync_copy(agg_sh.at[pl.ds(s * STRIPE, STRIPE)],
                        out.at[bt, pl.ds(s * STRIPE, STRIPE), c])
        plsc.subcore_barrier()


def _sc_aggregate(hall, srcg, etg, normg, dstg, zrows):
    hall2 = hall.reshape(BATCH * (R + 1) * NP * 2, D // 2)
    mesh = plsc.VectorSubcoreMesh(core_axis_name="c", subcore_axis_name="s",
                                  num_cores=NC, num_subcores=NS)
    agg = pl.kernel(
        _sc_body,
        out_type=jax.ShapeDtypeStruct((BATCH, NP, 2, D // 2), jnp.float32),
        mesh=mesh,
        scratch_types=[
            pltpu.VMEM((EPS,), jnp.int32),
            pltpu.VMEM((EPS,), jnp.int32),
            pltpu.VMEM((EPS,), jnp.float32),
            pltpu.VMEM((NCH, CH), jnp.int32),
            pltpu.VMEM((CH,), jnp.int32),
            pltpu.VMEM((CH, D // 2), jnp.float32),
            pltpu.VMEM_SHARED((N, D // 2), jnp.float32),
            pltpu.SemaphoreType.DMA,
        ],
    )(hall2, srcg, etg, normg, dstg, zrows)
    return agg.reshape(BATCH, NP, D)


def kernel(inputs, edge_index, edge_type, edge_norm,
           basis0, w_comp0, loop_w0, basis1, w_comp1, loop_w1):
    src = edge_index[0]
    dst = edge_index[1]
    srcg = src.reshape(NS, EPS)
    etg = edge_type.reshape(NS, EPS)
    normg = edge_norm.reshape(NS, EPS)
    dstg = dst.reshape(NS, NCH, CH)
    zrows = jnp.zeros((N, D // 2), jnp.float32)

    h = jnp.pad(inputs, ((0, 0), (0, NP - N), (0, 0)))
    for basis, w_comp, loop_w, relu in (
            (basis0, w_comp0, loop_w0, True),
            (basis1, w_comp1, loop_w1, False)):
        wall = _build_wall(w_comp, basis, loop_w)
        hall = _project(h, wall)
        agg = _sc_aggregate(hall, srcg, etg, normg, dstg, zrows)
        h = _finalize(agg, hall, relu)
    return h[:, :N, :]


# speedup vs baseline: 12.6787x; 12.6787x over previous
import functools

import jax
import jax.numpy as jnp
from jax import lax
from jax.experimental import pallas as pl
from jax.experimental.pallas import tpu as pltpu
from jax.experimental.pallas import tpu_sc as plsc

N = 10000
E = 160000
R = 16
NBASE = 4
D = 256
BATCH = 2
NP = 10240
BLK = 512

NC = 2
NS = 16
LANES = 16
EPS = E // NS
CH = 80
NCH = EPS // CH
STRIPE = NP // NS


def _wbuild_body(w_comp_ref, basis_ref, loop_ref, out_ref):
    w = jnp.dot(w_comp_ref[...], basis_ref[...],
                preferred_element_type=jnp.float32)
    out_ref[0:R, :] = w
    out_ref[R:R + 1, :] = loop_ref[...].reshape(1, D * D)


def _build_wall(w_comp, basis, loop_w):
    out = pl.pallas_call(
        _wbuild_body,
        out_shape=jax.ShapeDtypeStruct((R + 1, D * D), jnp.float32),
    )(w_comp, basis.reshape(NBASE, D * D), loop_w.reshape(D, D))
    return out.reshape(R + 1, D, D)


def _proj_body(h_ref, w_ref, out_ref):
    out_ref[0, 0] = jnp.dot(h_ref[0], w_ref[0],
                            preferred_element_type=jnp.float32)


def _project(h_pad, wall):
    grid = (BATCH, NP // BLK, R + 1)
    return pl.pallas_call(
        _proj_body,
        grid=grid,
        in_specs=[
            pl.BlockSpec((1, BLK, D), lambda b, n, r: (b, n, 0)),
            pl.BlockSpec((1, D, D), lambda b, n, r: (r, 0, 0)),
        ],
        out_specs=pl.BlockSpec((1, 1, BLK, D), lambda b, n, r: (b, r, n, 0)),
        out_shape=jax.ShapeDtypeStruct((BATCH, R + 1, NP, D), jnp.float32),
    )(h_pad, wall)


def _final_body(a0_ref, a1_ref, hall_ref, out_ref, *, relu):
    o = jnp.concatenate([a0_ref[0, 0], a1_ref[0, 0]], axis=-1)
    o = o + hall_ref[0, 0]
    if relu:
        o = jnp.maximum(o, 0.0)
    out_ref[0] = o


def _finalize(agg, hall, relu):
    grid = (BATCH, NP // BLK)
    return pl.pallas_call(
        functools.partial(_final_body, relu=relu),
        grid=grid,
        in_specs=[
            pl.BlockSpec((1, 1, BLK, D // 2), lambda b, n: (b, 0, n, 0)),
            pl.BlockSpec((1, 1, BLK, D // 2), lambda b, n: (b, 1, n, 0)),
            pl.BlockSpec((1, 1, BLK, D), lambda b, n: (b, R, n, 0)),
        ],
        out_specs=pl.BlockSpec((1, BLK, D), lambda b, n: (b, n, 0)),
        out_shape=jax.ShapeDtypeStruct((BATCH, NP, D), jnp.float32),
    )(agg, agg, hall)


SCH = 2000
NSCH = EPS // SCH
CPS = SCH // CH


def _sc_body(hall2, srcg, etg, normg, dstg, zrows, out,
             src_v, et_v, norm_v, dst_v, gidx_v, rows_v, agg_sh, sem):
    c = lax.axis_index("c")
    s = lax.axis_index("s")

    for bt in range(BATCH):
        pltpu.sync_copy(zrows.at[pl.ds(s * STRIPE, STRIPE)],
                        agg_sh.at[pl.ds(s * STRIPE, STRIPE)])
        plsc.subcore_barrier()

        def superchunk(g, carry):
            pltpu.sync_copy(srcg.at[s, g], src_v)
            pltpu.sync_copy(etg.at[s, g], et_v)
            pltpu.sync_copy(normg.at[s, g], norm_v)
            pltpu.sync_copy(dstg.at[s, g], dst_v)

            def chunk(k, carry2):
                base = k * CH
                for j in range(CH // LANES):
                    et = et_v[pl.ds(base + j * LANES, LANES)]
                    sr = src_v[pl.ds(base + j * LANES, LANES)]
                    gidx_v[pl.ds(j * LANES, LANES)] = (
                        ((bt * (R + 1) + et) * NP + sr) * 2 + c)
                pltpu.async_copy(hall2.at[gidx_v], rows_v, sem).wait()
                for e in range(CH):
                    nv = plsc.load_gather(
                        norm_v, [jnp.full((LANES,), base + e, jnp.int32)])
                    for q in range(D // 2 // LANES):
                        rows_v[e, pl.ds(q * LANES, LANES)] = (
                            rows_v[e, pl.ds(q * LANES, LANES)] * nv)
                pltpu.sync_copy(rows_v, agg_sh.at[dst_v.at[k]], add=True)
                return carry2

            lax.fori_loop(0, CPS, chunk, 0)
            return carry

        lax.fori_loop(0, NSCH, superchunk, 0)
        plsc.subcore_barrier()
        pltpu.sync_copy(agg_sh.at[pl.ds(s * STRIPE, STRIPE)],
                        out.at[bt, c, pl.ds(s * STRIPE, STRIPE)])
        plsc.subcore_barrier()


def _sc_aggregate(hall, srcg, etg, normg, dstg, zrows):
    hall2 = hall.reshape(BATCH * (R + 1) * NP * 2, D // 2)
    mesh = plsc.VectorSubcoreMesh(core_axis_name="c", subcore_axis_name="s",
                                  num_cores=NC, num_subcores=NS)
    agg = pl.kernel(
        _sc_body,
        out_type=jax.ShapeDtypeStruct((BATCH, 2, NP, D // 2), jnp.float32),
        mesh=mesh,
        compiler_params=pltpu.CompilerParams(needs_layout_passes=False),
        scratch_types=[
            pltpu.VMEM((SCH,), jnp.int32),
            pltpu.VMEM((SCH,), jnp.int32),
            pltpu.VMEM((SCH,), jnp.float32),
            pltpu.VMEM((CPS, CH), jnp.int32),
            pltpu.VMEM((CH,), jnp.int32),
            pltpu.VMEM((CH, D // 2), jnp.float32),
            pltpu.VMEM_SHARED((NP, D // 2), jnp.float32),
            pltpu.SemaphoreType.DMA,
        ],
    )(hall2, srcg, etg, normg, dstg, zrows)
    return agg


def kernel(inputs, edge_index, edge_type, edge_norm,
           basis0, w_comp0, loop_w0, basis1, w_comp1, loop_w1):
    src = edge_index[0]
    dst = edge_index[1]
    srcg = src.reshape(NS, NSCH, SCH)
    etg = edge_type.reshape(NS, NSCH, SCH)
    normg = edge_norm.reshape(NS, NSCH, SCH)
    dstg = dst.reshape(NS, NSCH, CPS, CH)
    zrows = jnp.zeros((NP, D // 2), jnp.float32)

    h = jnp.pad(inputs, ((0, 0), (0, NP - N), (0, 0)))
    for basis, w_comp, loop_w, relu in (
            (basis0, w_comp0, loop_w0, True),
            (basis1, w_comp1, loop_w1, False)):
        wall = _build_wall(w_comp, basis, loop_w)
        hall = _project(h, wall)
        agg = _sc_aggregate(hall, srcg, etg, normg, dstg, zrows)
        h = _finalize(agg, hall, relu)
    return h[:, :N, :]


# speedup vs baseline: 16.4241x; 1.2954x over previous
import functools

import jax
import jax.numpy as jnp
from jax import lax
from jax.experimental import pallas as pl
from jax.experimental.pallas import tpu as pltpu
from jax.experimental.pallas import tpu_sc as plsc

N = 10000
E = 160000
R = 16
NBASE = 4
D = 256
BATCH = 2
NP = 10240
BLK = 512

NC = 2
NS = 16
LANES = 16
EPS = E // NS
CH = 80
NCH = EPS // CH
STRIPE = NP // NS


def _wbuild_body(w_comp_ref, basis_ref, loop_ref, out_ref):
    w = jnp.dot(w_comp_ref[...], basis_ref[...],
                preferred_element_type=jnp.float32)
    out_ref[0:R, :] = w
    out_ref[R:R + 1, :] = loop_ref[...].reshape(1, D * D)


def _build_wall(w_comp, basis, loop_w):
    out = pl.pallas_call(
        _wbuild_body,
        out_shape=jax.ShapeDtypeStruct((R + 1, D * D), jnp.float32),
    )(w_comp, basis.reshape(NBASE, D * D), loop_w.reshape(D, D))
    return out.reshape(R + 1, D, D)


def _proj_body(h_ref, w_ref, lo_ref, hi_ref):
    r = jnp.dot(h_ref[0], w_ref[0], preferred_element_type=jnp.float32)
    lo_ref[0, 0] = r[:, :D // 2]
    hi_ref[0, 0] = r[:, D // 2:]


def _project(h_pad, wall):
    grid = (BATCH, NP // BLK, R + 1)
    half = jax.ShapeDtypeStruct((BATCH, R + 1, NP, D // 2), jnp.float32)
    return pl.pallas_call(
        _proj_body,
        grid=grid,
        in_specs=[
            pl.BlockSpec((1, BLK, D), lambda b, n, r: (b, n, 0)),
            pl.BlockSpec((1, D, D), lambda b, n, r: (r, 0, 0)),
        ],
        out_specs=[
            pl.BlockSpec((1, 1, BLK, D // 2), lambda b, n, r: (b, r, n, 0)),
            pl.BlockSpec((1, 1, BLK, D // 2), lambda b, n, r: (b, r, n, 0)),
        ],
        out_shape=[half, half],
    )(h_pad.astype(jnp.bfloat16), wall.astype(jnp.bfloat16))


def _final_body(a0_ref, a1_ref, s0_ref, s1_ref, out_ref, *, relu):
    o = jnp.concatenate([a0_ref[0, 0] + s0_ref[0, 0],
                         a1_ref[0, 0] + s1_ref[0, 0]], axis=-1)
    if relu:
        o = jnp.maximum(o, 0.0)
    out_ref[0] = o


def _finalize(agg, hall_lo, hall_hi, relu):
    grid = (BATCH, NP // BLK)
    return pl.pallas_call(
        functools.partial(_final_body, relu=relu),
        grid=grid,
        in_specs=[
            pl.BlockSpec((1, 1, BLK, D // 2), lambda b, n: (b, 0, n, 0)),
            pl.BlockSpec((1, 1, BLK, D // 2), lambda b, n: (b, 1, n, 0)),
            pl.BlockSpec((1, 1, BLK, D // 2), lambda b, n: (b, R, n, 0)),
            pl.BlockSpec((1, 1, BLK, D // 2), lambda b, n: (b, R, n, 0)),
        ],
        out_specs=pl.BlockSpec((1, BLK, D), lambda b, n: (b, n, 0)),
        out_shape=jax.ShapeDtypeStruct((BATCH, NP, D), jnp.float32),
    )(agg, agg, hall_lo, hall_hi)


SCH = 2000
NSCH = EPS // SCH
CPS = SCH // CH


def _sc_body(hall_lo, hall_hi, srcg, etg, normg, dstg, zrows, out,
             src_v, et_v, norm_v, dst_v, gidx_v, rows_v, agg_sh, sem):
    c = lax.axis_index("c")
    s = lax.axis_index("s")

    for bt in range(BATCH):
        pltpu.sync_copy(zrows.at[pl.ds(s * STRIPE, STRIPE)],
                        agg_sh.at[pl.ds(s * STRIPE, STRIPE)])
        plsc.subcore_barrier()

        def edge_sweep(table):
            def superchunk(g, carry):
                pltpu.sync_copy(srcg.at[s, g], src_v)
                pltpu.sync_copy(etg.at[s, g], et_v)
                pltpu.sync_copy(normg.at[s, g], norm_v)
                pltpu.sync_copy(dstg.at[s, g], dst_v)

                def chunk(k, carry2):
                    base = k * CH
                    for j in range(CH // LANES):
                        et = et_v[pl.ds(base + j * LANES, LANES)]
                        sr = src_v[pl.ds(base + j * LANES, LANES)]
                        gidx_v[pl.ds(j * LANES, LANES)] = (
                            (bt * (R + 1) + et) * NP + sr)
                    pltpu.async_copy(table.at[gidx_v], rows_v, sem).wait()
                    for e in range(CH):
                        nv = plsc.load_gather(
                            norm_v, [jnp.full((LANES,), base + e, jnp.int32)])
                        for q in range(D // 2 // LANES):
                            rows_v[e, pl.ds(q * LANES, LANES)] = (
                                rows_v[e, pl.ds(q * LANES, LANES)] * nv)
                    pltpu.sync_copy(rows_v, agg_sh.at[dst_v.at[k]], add=True)
                    return carry2

                lax.fori_loop(0, CPS, chunk, 0)
                return carry

            lax.fori_loop(0, NSCH, superchunk, 0)

        pl.when(c == 0)(lambda: edge_sweep(hall_lo))
        pl.when(c == 1)(lambda: edge_sweep(hall_hi))
        plsc.subcore_barrier()
        pltpu.sync_copy(agg_sh.at[pl.ds(s * STRIPE, STRIPE)],
                        out.at[bt, c, pl.ds(s * STRIPE, STRIPE)])
        plsc.subcore_barrier()


def _sc_aggregate(hall_lo, hall_hi, srcg, etg, normg, dstg, zrows):
    lo = hall_lo.reshape(BATCH * (R + 1) * NP, D // 2)
    hi = hall_hi.reshape(BATCH * (R + 1) * NP, D // 2)
    mesh = plsc.VectorSubcoreMesh(core_axis_name="c", subcore_axis_name="s",
                                  num_cores=NC, num_subcores=NS)
    agg = pl.kernel(
        _sc_body,
        out_type=jax.ShapeDtypeStruct((BATCH, 2, NP, D // 2), jnp.float32),
        mesh=mesh,
        compiler_params=pltpu.CompilerParams(needs_layout_passes=False),
        scratch_types=[
            pltpu.VMEM((SCH,), jnp.int32),
            pltpu.VMEM((SCH,), jnp.int32),
            pltpu.VMEM((SCH,), jnp.float32),
            pltpu.VMEM((CPS, CH), jnp.int32),
            pltpu.VMEM((CH,), jnp.int32),
            pltpu.VMEM((CH, D // 2), jnp.float32),
            pltpu.VMEM_SHARED((NP, D // 2), jnp.float32),
            pltpu.SemaphoreType.DMA,
        ],
    )(lo, hi, srcg, etg, normg, dstg, zrows)
    return agg


def kernel(inputs, edge_index, edge_type, edge_norm,
           basis0, w_comp0, loop_w0, basis1, w_comp1, loop_w1):
    src = edge_index[0]
    dst = edge_index[1]
    srcg = src.reshape(NS, NSCH, SCH)
    etg = edge_type.reshape(NS, NSCH, SCH)
    normg = edge_norm.reshape(NS, NSCH, SCH)
    dstg = dst.reshape(NS, NSCH, CPS, CH)
    zrows = jnp.zeros((NP, D // 2), jnp.float32)

    h = jnp.pad(inputs, ((0, 0), (0, NP - N), (0, 0)))
    for basis, w_comp, loop_w, relu in (
            (basis0, w_comp0, loop_w0, True),
            (basis1, w_comp1, loop_w1, False)):
        wall = _build_wall(w_comp, basis, loop_w)
        hall_lo, hall_hi = _project(h, wall)
        agg = _sc_aggregate(hall_lo, hall_hi, srcg, etg, normg, dstg, zrows)
        h = _finalize(agg, hall_lo, hall_hi, relu)
    return h[:, :N, :]
